# baseline (speedup 1.0000x reference)
import functools

import jax
import jax.numpy as jnp
from jax import lax
from jax.experimental import pallas as pl
from jax.experimental.pallas import tpu as pltpu
from jax.experimental.pallas import tpu_sc as plsc

N = 10000
E = 320000
F_IN = 128
H = 128
C = 64

NPAD = 10240
NW = 32
EPW = E // NW
CHW = 128
CH = 80
EPW_PAD = CH * CHW
DCH = -(-(2 * EPW) // CHW)
DPW_PAD = DCH * CHW

_mesh = plsc.VectorSubcoreMesh(core_axis_name="c", subcore_axis_name="s")


@functools.partial(
    pl.kernel,
    mesh=_mesh,
    out_type=jax.ShapeDtypeStruct((2, 2 * NPAD), jnp.float32),
    scratch_types=[
        pltpu.VMEM((DCH, CHW), jnp.int32),
        pltpu.VMEM((CHW,), jnp.float32),
        pltpu.VMEM_SHARED((2 * NPAD,), jnp.float32),
    ],
)
def _sc_degrees(idx_hbm, ones_hbm, zeros_hbm, out_hbm, idx_v, ones_v, acc_sh):
    c = lax.axis_index("c")
    s = lax.axis_index("s")
    wid = s * 2 + c
    sl = (2 * NPAD) // 16
    pltpu.sync_copy(zeros_hbm.at[pl.ds(s * sl, sl)], acc_sh.at[pl.ds(s * sl, sl)])
    pltpu.sync_copy(ones_hbm, ones_v)
    pltpu.sync_copy(idx_hbm.at[wid], idx_v)
    plsc.subcore_barrier()

    def body(j, carry):
        pltpu.sync_copy(ones_v, acc_sh.at[idx_v.at[j]], add=True)
        return carry

    lax.fori_loop(0, DCH, body, 0)
    plsc.subcore_barrier()
    pltpu.sync_copy(acc_sh.at[pl.ds(s * sl, sl)], out_hbm.at[c, pl.ds(s * sl, sl)])


def _make_sc_agg(F):
    @functools.partial(
        pl.kernel,
        mesh=_mesh,
        out_type=jax.ShapeDtypeStruct((2, NPAD, F), jnp.float32),
        scratch_types=[
            pltpu.VMEM((CH // 2, CHW), jnp.int32),
            pltpu.VMEM((CH // 2, CHW), jnp.int32),
            pltpu.VMEM((CHW, F), jnp.float32),
            pltpu.VMEM((CHW, F), jnp.float32),
            pltpu.VMEM_SHARED((NPAD, F), jnp.float32),
            pltpu.SemaphoreType.DMA,
            pltpu.SemaphoreType.DMA,
            pltpu.SemaphoreType.DMA,
            pltpu.SemaphoreType.DMA,
        ],
    )
    def _sc_agg(table_hbm, sidx_hbm, didx_hbm, zeros_hbm, out_hbm,
                sidx_v, didx_v, rows0_v, rows1_v, acc_sh,
                g0, g1, s0, s1):
        c = lax.axis_index("c")
        s = lax.axis_index("s")
        wid = s * 2 + c
        rs = NPAD // 16
        cpp = CH // 2
        pltpu.sync_copy(zeros_hbm.at[pl.ds(s * rs, rs)], acc_sh.at[pl.ds(s * rs, rs)])
        plsc.subcore_barrier()

        for p in range(2):
            pltpu.sync_copy(sidx_hbm.at[wid, pl.ds(p * cpp, cpp)], sidx_v)
            pltpu.sync_copy(didx_hbm.at[wid, pl.ds(p * cpp, cpp)], didx_v)

            pltpu.async_copy(table_hbm.at[sidx_v.at[0]], rows0_v, g0)
            pltpu.async_copy(table_hbm.at[sidx_v.at[1]], rows1_v, g1)

            def body(j2, carry):
                j = j2 * 2
                pltpu.make_async_copy(table_hbm.at[sidx_v.at[j]], rows0_v,
                                      g0).wait()
                pltpu.async_copy(rows0_v, acc_sh.at[didx_v.at[j]], s0,
                                 add=True)
                pltpu.make_async_copy(table_hbm.at[sidx_v.at[j + 1]], rows1_v,
                                      g1).wait()
                pltpu.async_copy(rows1_v, acc_sh.at[didx_v.at[j + 1]], s1,
                                 add=True)
                pltpu.make_async_copy(rows0_v, acc_sh.at[didx_v.at[j]],
                                      s0).wait()

                @pl.when(j2 < cpp // 2 - 1)
                def _():
                    pltpu.async_copy(table_hbm.at[sidx_v.at[j + 2]], rows0_v,
                                     g0)

                pltpu.make_async_copy(rows1_v, acc_sh.at[didx_v.at[j + 1]],
                                      s1).wait()

                @pl.when(j2 < cpp // 2 - 1)
                def _():
                    pltpu.async_copy(table_hbm.at[sidx_v.at[j + 3]], rows1_v,
                                     g1)
                return carry

            lax.fori_loop(0, cpp // 2, body, 0)
        plsc.subcore_barrier()
        pltpu.sync_copy(acc_sh.at[pl.ds(s * rs, rs)],
                        out_hbm.at[c, pl.ds(s * rs, rs)])

    return _sc_agg


_sc_agg_h = _make_sc_agg(H)


def _norm_cols(deg_ref):
    d = deg_ref[...]
    deg_out = jnp.maximum(d[:, 0:1] + d[:, 2:3], 1.0)
    deg_in = jnp.maximum(d[:, 1:2] + d[:, 3:4], 1.0)
    return lax.rsqrt(deg_out)[0:N], lax.rsqrt(deg_in)[0:N]


def _tc_prep_body(x_ref, w_ref, deg_ref, o_ref):
    norm_src, _ = _norm_cols(deg_ref)
    x = x_ref[...] * norm_src
    o_ref[...] = jnp.dot(x, w_ref[...], preferred_element_type=jnp.float32)


def _tc_mid_body(aggp_ref, deg_ref, b1_ref, o_ref):
    norm_src, norm_dst = _norm_cols(deg_ref)
    agg = aggp_ref[0, 0:N, :] + aggp_ref[1, 0:N, :]
    h = jnp.maximum(agg * norm_dst + b1_ref[...], 0.0)
    o_ref[...] = h * norm_src


def _tc_fin_body(aggp_ref, deg_ref, w2_ref, b2_ref, o_ref):
    _, norm_dst = _norm_cols(deg_ref)
    agg = aggp_ref[0, 0:N, :] + aggp_ref[1, 0:N, :]
    o_ref[...] = (jnp.dot(agg, w2_ref[...], preferred_element_type=jnp.float32)
                  * norm_dst + b2_ref[...])


def kernel(n_feats, edge_index, W1, b1, W2, b2):
    src = edge_index[0].astype(jnp.int32)
    dst = edge_index[1].astype(jnp.int32)

    srcw = src.reshape(NW, EPW)
    dstw = dst.reshape(NW, EPW)
    padn = EPW_PAD - EPW
    pad_src = (jnp.arange(padn, dtype=jnp.int32) * 89) % N
    pad_dst = N + (jnp.arange(padn, dtype=jnp.int32) % (NPAD - N))
    src_p = jnp.concatenate(
        [srcw, jnp.broadcast_to(pad_src, (NW, padn))], axis=1
    ).reshape(NW, CH, CHW)
    dst_p = jnp.concatenate(
        [dstw, jnp.broadcast_to(pad_dst, (NW, padn))], axis=1
    ).reshape(NW, CH, CHW)

    dpadn = DPW_PAD - 2 * EPW
    pad_deg = 2 * N + ((jnp.arange(dpadn, dtype=jnp.int32) * 3) % (2 * (NPAD - N)))
    deg_idx = jnp.concatenate(
        [2 * srcw, 2 * dstw + 1, jnp.broadcast_to(pad_deg, (NW, dpadn))], axis=1
    ).reshape(NW, DCH, CHW)

    ones_chunk = jnp.ones((CHW,), jnp.float32)
    zeros_deg = jnp.zeros((2 * NPAD,), jnp.float32)
    zeros_h = jnp.zeros((NPAD, H), jnp.float32)

    deg2 = _sc_degrees(deg_idx, ones_chunk, zeros_deg)
    deg_cols = deg2.reshape(2, NPAD, 2).transpose(1, 0, 2).reshape(NPAD, 4)

    xw = pl.pallas_call(
        _tc_prep_body,
        out_shape=jax.ShapeDtypeStruct((N, H), jnp.float32),
    )(n_feats, W1, deg_cols)

    agg1p = _sc_agg_h(xw, src_p, dst_p, zeros_h)

    hs = pl.pallas_call(
        _tc_mid_body,
        out_shape=jax.ShapeDtypeStruct((N, H), jnp.float32),
    )(agg1p, deg_cols, b1.reshape(1, H))

    agg2p = _sc_agg_h(hs, src_p, dst_p, zeros_h)

    out = pl.pallas_call(
        _tc_fin_body,
        out_shape=jax.ShapeDtypeStruct((N, C), jnp.float32),
    )(agg2p, deg_cols, W2, b2.reshape(1, C))

    return out


# speedup vs baseline: 1.0751x; 1.0751x over previous
import functools

import jax
import jax.numpy as jnp
from jax import lax
from jax.experimental import pallas as pl
from jax.experimental.pallas import tpu as pltpu
from jax.experimental.pallas import tpu_sc as plsc

N = 10000
E = 320000
F_IN = 128
H = 128
C = 64

NPAD = 10240
NW = 32
EPW = E // NW
CHW = 128
CH = 80
EPW_PAD = CH * CHW
DCH = -(-(2 * EPW) // CHW)
DPW_PAD = DCH * CHW

_mesh = plsc.VectorSubcoreMesh(core_axis_name="c", subcore_axis_name="s")


@functools.partial(
    pl.kernel,
    mesh=_mesh,
    out_type=jax.ShapeDtypeStruct((2, 2 * NPAD), jnp.float32),
    scratch_types=[
        pltpu.VMEM((DCH, CHW), jnp.int32),
        pltpu.VMEM((CHW,), jnp.float32),
        pltpu.VMEM_SHARED((2 * NPAD,), jnp.float32),
    ],
)
def _sc_degrees(idx_hbm, ones_hbm, zeros_hbm, out_hbm, idx_v, ones_v, acc_sh):
    c = lax.axis_index("c")
    s = lax.axis_index("s")
    wid = s * 2 + c
    sl = (2 * NPAD) // 16
    pltpu.sync_copy(zeros_hbm.at[pl.ds(s * sl, sl)], acc_sh.at[pl.ds(s * sl, sl)])
    pltpu.sync_copy(ones_hbm, ones_v)
    pltpu.sync_copy(idx_hbm.at[wid], idx_v)
    plsc.subcore_barrier()

    def body(j, carry):
        pltpu.sync_copy(ones_v, acc_sh.at[idx_v.at[j]], add=True)
        return carry

    lax.fori_loop(0, DCH, body, 0)
    plsc.subcore_barrier()
    pltpu.sync_copy(acc_sh.at[pl.ds(s * sl, sl)], out_hbm.at[c, pl.ds(s * sl, sl)])


def _make_sc_agg(F):
    @functools.partial(
        pl.kernel,
        mesh=_mesh,
        out_type=jax.ShapeDtypeStruct((2, NPAD, F), jnp.float32),
        scratch_types=[
            pltpu.VMEM((CH // 2, CHW), jnp.int32),
            pltpu.VMEM((CH // 2, CHW), jnp.int32),
            pltpu.VMEM((CHW, F), jnp.float32),
            pltpu.VMEM((CHW, F), jnp.float32),
            pltpu.VMEM_SHARED((NPAD, F), jnp.float32),
            pltpu.SemaphoreType.DMA,
            pltpu.SemaphoreType.DMA,
        ],
    )
    def _sc_agg(table_hbm, sidx_hbm, didx_hbm, zeros_hbm, out_hbm,
                sidx_v, didx_v, rows0_v, rows1_v, acc_sh, g0, g1):
        c = lax.axis_index("c")
        s = lax.axis_index("s")
        wid = s * 2 + c
        rs = NPAD // 16
        cpp = CH // 2
        pltpu.sync_copy(zeros_hbm.at[pl.ds(s * rs, rs)], acc_sh.at[pl.ds(s * rs, rs)])
        plsc.subcore_barrier()

        for p in range(2):
            pltpu.sync_copy(sidx_hbm.at[wid, pl.ds(p * cpp, cpp)], sidx_v)
            pltpu.sync_copy(didx_hbm.at[wid, pl.ds(p * cpp, cpp)], didx_v)

            pltpu.async_copy(table_hbm.at[sidx_v.at[0]], rows0_v, g0)

            def body(j2, carry):
                j = j2 * 2
                pltpu.make_async_copy(table_hbm.at[sidx_v.at[j]], rows0_v,
                                      g0).wait()
                pltpu.async_copy(table_hbm.at[sidx_v.at[j + 1]], rows1_v, g1)
                pltpu.sync_copy(rows0_v, acc_sh.at[didx_v.at[j]], add=True)
                pltpu.make_async_copy(table_hbm.at[sidx_v.at[j + 1]], rows1_v,
                                      g1).wait()

                @pl.when(j2 < cpp // 2 - 1)
                def _():
                    pltpu.async_copy(table_hbm.at[sidx_v.at[j + 2]], rows0_v,
                                     g0)

                pltpu.sync_copy(rows1_v, acc_sh.at[didx_v.at[j + 1]], add=True)
                return carry

            lax.fori_loop(0, cpp // 2, body, 0)
        plsc.subcore_barrier()
        pltpu.sync_copy(acc_sh.at[pl.ds(s * rs, rs)],
                        out_hbm.at[c, pl.ds(s * rs, rs)])

    return _sc_agg


_sc_agg_h = _make_sc_agg(H)


def _norm_cols(deg_ref):
    d = deg_ref[...]
    deg_out = jnp.maximum(d[:, 0:1] + d[:, 2:3], 1.0)
    deg_in = jnp.maximum(d[:, 1:2] + d[:, 3:4], 1.0)
    return lax.rsqrt(deg_out)[0:N], lax.rsqrt(deg_in)[0:N]


def _tc_prep_body(x_ref, w_ref, deg_ref, o_ref):
    norm_src, _ = _norm_cols(deg_ref)
    x = x_ref[...] * norm_src
    o_ref[...] = jnp.dot(x, w_ref[...], preferred_element_type=jnp.float32)


def _tc_mid_body(aggp_ref, deg_ref, b1_ref, o_ref):
    norm_src, norm_dst = _norm_cols(deg_ref)
    agg = aggp_ref[0, 0:N, :] + aggp_ref[1, 0:N, :]
    h = jnp.maximum(agg * norm_dst + b1_ref[...], 0.0)
    o_ref[...] = h * norm_src


def _tc_fin_body(aggp_ref, deg_ref, w2_ref, b2_ref, o_ref):
    _, norm_dst = _norm_cols(deg_ref)
    agg = aggp_ref[0, 0:N, :] + aggp_ref[1, 0:N, :]
    o_ref[...] = (jnp.dot(agg, w2_ref[...], preferred_element_type=jnp.float32)
                  * norm_dst + b2_ref[...])


def kernel(n_feats, edge_index, W1, b1, W2, b2):
    src = edge_index[0].astype(jnp.int32)
    dst = edge_index[1].astype(jnp.int32)

    srcw = src.reshape(NW, EPW)
    dstw = dst.reshape(NW, EPW)
    padn = EPW_PAD - EPW
    pad_src = (jnp.arange(padn, dtype=jnp.int32) * 89) % N
    pad_dst = N + (jnp.arange(padn, dtype=jnp.int32) % (NPAD - N))
    src_p = jnp.concatenate(
        [srcw, jnp.broadcast_to(pad_src, (NW, padn))], axis=1
    ).reshape(NW, CH, CHW)
    dst_p = jnp.concatenate(
        [dstw, jnp.broadcast_to(pad_dst, (NW, padn))], axis=1
    ).reshape(NW, CH, CHW)

    dpadn = DPW_PAD - 2 * EPW
    pad_deg = 2 * N + ((jnp.arange(dpadn, dtype=jnp.int32) * 3) % (2 * (NPAD - N)))
    deg_idx = jnp.concatenate(
        [2 * srcw, 2 * dstw + 1, jnp.broadcast_to(pad_deg, (NW, dpadn))], axis=1
    ).reshape(NW, DCH, CHW)

    ones_chunk = jnp.ones((CHW,), jnp.float32)
    zeros_deg = jnp.zeros((2 * NPAD,), jnp.float32)
    zeros_h = jnp.zeros((NPAD, H), jnp.float32)

    deg2 = _sc_degrees(deg_idx, ones_chunk, zeros_deg)
    deg_cols = deg2.reshape(2, NPAD, 2).transpose(1, 0, 2).reshape(NPAD, 4)

    xw = pl.pallas_call(
        _tc_prep_body,
        out_shape=jax.ShapeDtypeStruct((N, H), jnp.float32),
    )(n_feats, W1, deg_cols)

    agg1p = _sc_agg_h(xw, src_p, dst_p, zeros_h)

    hs = pl.pallas_call(
        _tc_mid_body,
        out_shape=jax.ShapeDtypeStruct((N, H), jnp.float32),
    )(agg1p, deg_cols, b1.reshape(1, H))

    agg2p = _sc_agg_h(hs, src_p, dst_p, zeros_h)

    out = pl.pallas_call(
        _tc_fin_body,
        out_shape=jax.ShapeDtypeStruct((N, C), jnp.float32),
    )(agg2p, deg_cols, W2, b2.reshape(1, C))

    return out


# speedup vs baseline: 1.2103x; 1.1258x over previous
import functools

import jax
import jax.numpy as jnp
from jax import lax
from jax.experimental import pallas as pl
from jax.experimental.pallas import tpu as pltpu
from jax.experimental.pallas import tpu_sc as plsc

N = 10000
E = 320000
F_IN = 128
H = 128
C = 64

NPAD = 10240
NW = 32
CHW = 125
NCH = E // CHW
WCH = NCH // NW
PH0 = 40

_mesh = plsc.VectorSubcoreMesh(core_axis_name="c", subcore_axis_name="s")


@functools.partial(
    pl.kernel,
    mesh=_mesh,
    out_type=jax.ShapeDtypeStruct((2, 2, 1, NPAD), jnp.float32),
    scratch_types=[
        pltpu.VMEM((2, WCH, CHW), jnp.int32),
        pltpu.VMEM((CHW,), jnp.float32),
        pltpu.VMEM_SHARED((2 * NPAD,), jnp.float32),
    ],
)
def _sc_degrees(idx_hbm, ones_hbm, zeros_hbm, out_hbm, idx_v, ones_v, acc_sh):
    c = lax.axis_index("c")
    s = lax.axis_index("s")
    wid = s * 2 + c
    sl = (2 * NPAD) // 16
    pltpu.sync_copy(zeros_hbm.at[pl.ds(s * sl, sl)], acc_sh.at[pl.ds(s * sl, sl)])
    pltpu.sync_copy(ones_hbm, ones_v)
    pltpu.sync_copy(idx_hbm.at[0, pl.ds(wid * WCH, WCH)],
                    idx_v.at[0])
    pltpu.sync_copy(idx_hbm.at[1, pl.ds(wid * WCH, WCH)],
                    idx_v.at[1])
    plsc.subcore_barrier()

    for h in range(2):
        def body(j, carry, h=h):
            pltpu.sync_copy(ones_v, acc_sh.at[idx_v.at[h, j]], add=True)
            return carry

        lax.fori_loop(0, WCH, body, 0)
    plsc.subcore_barrier()
    pltpu.sync_copy(
        acc_sh.at[pl.ds(s * sl, sl)],
        out_hbm.at[c, s // 8, 0, pl.ds((s % 8) * sl, sl)])


def _make_sc_agg(F):
    @functools.partial(
        pl.kernel,
        mesh=_mesh,
        out_type=jax.ShapeDtypeStruct((2, NPAD, F), jnp.float32),
        scratch_types=[
            pltpu.VMEM((PH0, CHW), jnp.int32),
            pltpu.VMEM((PH0, CHW), jnp.int32),
            pltpu.VMEM((CHW, F), jnp.float32),
            pltpu.VMEM((CHW, F), jnp.float32),
            pltpu.VMEM_SHARED((NPAD, F), jnp.float32),
            pltpu.SemaphoreType.DMA,
            pltpu.SemaphoreType.DMA,
        ],
    )
    def _sc_agg(table_hbm, sidx_hbm, didx_hbm, zeros_hbm, out_hbm,
                sidx_v, didx_v, rows0_v, rows1_v, acc_sh, g0, g1):
        c = lax.axis_index("c")
        s = lax.axis_index("s")
        wid = s * 2 + c
        rs = NPAD // 16
        pltpu.sync_copy(zeros_hbm.at[pl.ds(s * rs, rs)], acc_sh.at[pl.ds(s * rs, rs)])
        plsc.subcore_barrier()

        def ring(cpp):
            pltpu.async_copy(table_hbm.at[sidx_v.at[0]], rows0_v, g0)

            def body(j2, carry):
                j = j2 * 2
                pltpu.make_async_copy(table_hbm.at[sidx_v.at[j]], rows0_v,
                                      g0).wait()
                pltpu.async_copy(table_hbm.at[sidx_v.at[j + 1]], rows1_v, g1)
                pltpu.sync_copy(rows0_v, acc_sh.at[didx_v.at[j]], add=True)
                pltpu.make_async_copy(table_hbm.at[sidx_v.at[j + 1]], rows1_v,
                                      g1).wait()

                @pl.when(j2 < cpp // 2 - 1)
                def _():
                    pltpu.async_copy(table_hbm.at[sidx_v.at[j + 2]], rows0_v,
                                     g0)

                pltpu.sync_copy(rows1_v, acc_sh.at[didx_v.at[j + 1]], add=True)
                return carry

            lax.fori_loop(0, cpp // 2, body, 0)

        for p in range(WCH // PH0):
            base = wid * WCH + p * PH0
            pltpu.sync_copy(sidx_hbm.at[pl.ds(base, PH0)], sidx_v)
            pltpu.sync_copy(didx_hbm.at[pl.ds(base, PH0)], didx_v)
            ring(PH0)

        plsc.subcore_barrier()
        pltpu.sync_copy(acc_sh.at[pl.ds(s * rs, rs)],
                        out_hbm.at[c, pl.ds(s * rs, rs)])

    return _sc_agg


_sc_agg_h = _make_sc_agg(H)


def _norm_col(deg_ref, which):
    d = deg_ref[0, which, 0, :] + deg_ref[1, which, 0, :]
    nrm = lax.rsqrt(jnp.maximum(d, 1.0))
    nb = jnp.broadcast_to(nrm.reshape(1, NPAD), (8, NPAD))
    return lax.transpose(nb, (1, 0))[0:N, 0:1]


def _tc_prep_body(x_ref, w_ref, deg_ref, o_ref):
    norm_src = _norm_col(deg_ref, 0)
    x = x_ref[...] * norm_src
    o_ref[...] = jnp.dot(x, w_ref[...], preferred_element_type=jnp.float32)


def _tc_mid_body(aggp_ref, deg_ref, b1_ref, o_ref):
    norm_src = _norm_col(deg_ref, 0)
    norm_dst = _norm_col(deg_ref, 1)
    agg = aggp_ref[0, 0:N, :] + aggp_ref[1, 0:N, :]
    h = jnp.maximum(agg * norm_dst + b1_ref[...], 0.0)
    o_ref[...] = h * norm_src


def _tc_fin_body(aggp_ref, deg_ref, w2_ref, b2_ref, o_ref):
    norm_dst = _norm_col(deg_ref, 1)
    agg = aggp_ref[0, 0:N, :] + aggp_ref[1, 0:N, :]
    o_ref[...] = (jnp.dot(agg, w2_ref[...], preferred_element_type=jnp.float32)
                  * norm_dst + b2_ref[...])


def kernel(n_feats, edge_index, W1, b1, W2, b2):
    ei = edge_index.astype(jnp.int32)
    src_c = ei[0].reshape(NCH, CHW)
    dst_c = ei[1].reshape(NCH, CHW)
    deg_idx = (ei + jnp.array([[0], [NPAD]], jnp.int32)).reshape(2, NCH, CHW)

    ones_chunk = jnp.ones((CHW,), jnp.float32)
    zeros_deg = jnp.zeros((2 * NPAD,), jnp.float32)
    zeros_h = jnp.zeros((NPAD, H), jnp.float32)

    degp = _sc_degrees(deg_idx, ones_chunk, zeros_deg)

    xw = pl.pallas_call(
        _tc_prep_body,
        out_shape=jax.ShapeDtypeStruct((N, H), jnp.float32),
    )(n_feats, W1, degp)

    agg1p = _sc_agg_h(xw, src_c, dst_c, zeros_h)

    hs = pl.pallas_call(
        _tc_mid_body,
        out_shape=jax.ShapeDtypeStruct((N, H), jnp.float32),
    )(agg1p, degp, b1.reshape(1, H))

    agg2p = _sc_agg_h(hs, src_c, dst_c, zeros_h)

    out = pl.pallas_call(
        _tc_fin_body,
        out_shape=jax.ShapeDtypeStruct((N, C), jnp.float32),
    )(agg2p, degp, W2, b2.reshape(1, C))

    return out


# speedup vs baseline: 1.3073x; 1.0801x over previous
import functools

import jax
import jax.numpy as jnp
from jax import lax
from jax.experimental import pallas as pl
from jax.experimental.pallas import tpu as pltpu
from jax.experimental.pallas import tpu_sc as plsc

N = 10000
E = 320000
F_IN = 128
H = 128
C = 64

NPAD = 10240
NW = 32
CHW = 125
NCH = E // CHW
WCH = NCH // NW
PH0 = 40

_mesh = plsc.VectorSubcoreMesh(core_axis_name="c", subcore_axis_name="s")


@functools.partial(
    pl.kernel,
    mesh=_mesh,
    out_type=jax.ShapeDtypeStruct((2, 2, 1, NPAD), jnp.float32),
    scratch_types=[
        pltpu.VMEM((2, WCH, CHW), jnp.int32),
        pltpu.VMEM((CHW,), jnp.float32),
        pltpu.VMEM_SHARED((2 * NPAD,), jnp.float32),
    ],
)
def _sc_degrees(idx_hbm, ones_hbm, zeros_hbm, out_hbm, idx_v, ones_v, acc_sh):
    c = lax.axis_index("c")
    s = lax.axis_index("s")
    wid = s * 2 + c
    sl = (2 * NPAD) // 16
    pltpu.sync_copy(zeros_hbm.at[pl.ds(s * sl, sl)], acc_sh.at[pl.ds(s * sl, sl)])
    pltpu.sync_copy(ones_hbm, ones_v)
    pltpu.sync_copy(idx_hbm.at[0, pl.ds(wid * WCH, WCH)],
                    idx_v.at[0])
    pltpu.sync_copy(idx_hbm.at[1, pl.ds(wid * WCH, WCH)],
                    idx_v.at[1])
    plsc.subcore_barrier()

    for h in range(2):
        def body(j, carry, h=h):
            pltpu.sync_copy(ones_v, acc_sh.at[idx_v.at[h, j]], add=True)
            return carry

        lax.fori_loop(0, WCH, body, 0)
    plsc.subcore_barrier()
    pltpu.sync_copy(
        acc_sh.at[pl.ds(s * sl, sl)],
        out_hbm.at[c, s // 8, 0, pl.ds((s % 8) * sl, sl)])


def _make_sc_agg(F, ph, tc_tiling=True):
    kw = {}
    if not tc_tiling:
        kw["compiler_params"] = pltpu.CompilerParams(use_tc_tiling_on_sc=False)

    @functools.partial(
        pl.kernel,
        mesh=_mesh,
        out_type=jax.ShapeDtypeStruct((2, NPAD, F), jnp.float32),
        scratch_types=[
            pltpu.VMEM((ph, CHW), jnp.int32),
            pltpu.VMEM((ph, CHW), jnp.int32),
            pltpu.VMEM((CHW, F), jnp.float32),
            pltpu.VMEM((CHW, F), jnp.float32),
            pltpu.VMEM_SHARED((NPAD, F), jnp.float32),
            pltpu.SemaphoreType.DMA,
            pltpu.SemaphoreType.DMA,
        ],
        **kw,
    )
    def _sc_agg(table_hbm, sidx_hbm, didx_hbm, zeros_hbm, out_hbm,
                sidx_v, didx_v, rows0_v, rows1_v, acc_sh, g0, g1):
        c = lax.axis_index("c")
        s = lax.axis_index("s")
        wid = s * 2 + c
        rs = NPAD // 16
        pltpu.sync_copy(zeros_hbm.at[pl.ds(s * rs, rs)], acc_sh.at[pl.ds(s * rs, rs)])
        plsc.subcore_barrier()

        def ring(cpp):
            pltpu.async_copy(table_hbm.at[sidx_v.at[0]], rows0_v, g0)

            def body(j2, carry):
                j = j2 * 2
                pltpu.make_async_copy(table_hbm.at[sidx_v.at[j]], rows0_v,
                                      g0).wait()
                pltpu.async_copy(table_hbm.at[sidx_v.at[j + 1]], rows1_v, g1)
                pltpu.sync_copy(rows0_v, acc_sh.at[didx_v.at[j]], add=True)
                pltpu.make_async_copy(table_hbm.at[sidx_v.at[j + 1]], rows1_v,
                                      g1).wait()

                @pl.when(j2 < cpp // 2 - 1)
                def _():
                    pltpu.async_copy(table_hbm.at[sidx_v.at[j + 2]], rows0_v,
                                     g0)

                pltpu.sync_copy(rows1_v, acc_sh.at[didx_v.at[j + 1]], add=True)
                return carry

            lax.fori_loop(0, cpp // 2, body, 0)

        for p in range(WCH // ph):
            base = wid * WCH + p * ph
            pltpu.sync_copy(sidx_hbm.at[pl.ds(base, ph)], sidx_v)
            pltpu.sync_copy(didx_hbm.at[pl.ds(base, ph)], didx_v)
            ring(ph)

        plsc.subcore_barrier()
        pltpu.sync_copy(acc_sh.at[pl.ds(s * rs, rs)],
                        out_hbm.at[c, pl.ds(s * rs, rs)])

    return _sc_agg


_sc_agg_h = _make_sc_agg(H, PH0)
_sc_agg_c = _make_sc_agg(C, WCH, tc_tiling=False)


def _norm_col(deg_ref, which):
    d = deg_ref[0, which, 0, :] + deg_ref[1, which, 0, :]
    nrm = lax.rsqrt(jnp.maximum(d, 1.0))
    nb = jnp.broadcast_to(nrm.reshape(1, NPAD), (8, NPAD))
    return lax.transpose(nb, (1, 0))[0:N, 0:1]


def _tc_prep_body(x_ref, w_ref, deg_ref, o_ref):
    norm_src = _norm_col(deg_ref, 0)
    x = x_ref[...] * norm_src
    o_ref[...] = jnp.dot(x, w_ref[...], preferred_element_type=jnp.float32)


def _tc_mid_body(aggp_ref, deg_ref, b1_ref, w2_ref, o_ref):
    norm_src = _norm_col(deg_ref, 0)
    norm_dst = _norm_col(deg_ref, 1)
    agg = aggp_ref[0, 0:N, :] + aggp_ref[1, 0:N, :]
    h = jnp.maximum(agg * norm_dst + b1_ref[...], 0.0)
    o_ref[...] = jnp.dot(h * norm_src, w2_ref[...],
                         preferred_element_type=jnp.float32)


def _tc_fin_body(aggp_ref, deg_ref, b2_ref, o_ref):
    norm_dst = _norm_col(deg_ref, 1)
    agg = aggp_ref[0, 0:N, :] + aggp_ref[1, 0:N, :]
    o_ref[...] = agg * norm_dst + b2_ref[...]


def kernel(n_feats, edge_index, W1, b1, W2, b2):
    ei = edge_index.astype(jnp.int32)
    src_c = ei[0].reshape(NCH, CHW)
    dst_c = ei[1].reshape(NCH, CHW)
    deg_idx = (ei + jnp.array([[0], [NPAD]], jnp.int32)).reshape(2, NCH, CHW)

    ones_chunk = jnp.ones((CHW,), jnp.float32)
    zeros_deg = jnp.zeros((2 * NPAD,), jnp.float32)
    zeros_h = jnp.zeros((NPAD, H), jnp.float32)
    zeros_c = jnp.zeros((NPAD, C), jnp.float32)

    degp = _sc_degrees(deg_idx, ones_chunk, zeros_deg)

    xw = pl.pallas_call(
        _tc_prep_body,
        out_shape=jax.ShapeDtypeStruct((N, H), jnp.float32),
    )(n_feats, W1, degp)

    agg1p = _sc_agg_h(xw, src_c, dst_c, zeros_h)

    y = pl.pallas_call(
        _tc_mid_body,
        out_shape=jax.ShapeDtypeStruct((N, C), jnp.float32),
    )(agg1p, degp, b1.reshape(1, H), W2)

    agg2p = _sc_agg_c(y, src_c, dst_c, zeros_c)

    out = pl.pallas_call(
        _tc_fin_body,
        out_shape=jax.ShapeDtypeStruct((N, C), jnp.float32),
    )(agg2p, degp, b2.reshape(1, C))

    return out


# speedup vs baseline: 1.3162x; 1.0068x over previous
import functools

import jax
import jax.numpy as jnp
from jax import lax
from jax.experimental import pallas as pl
from jax.experimental.pallas import tpu as pltpu
from jax.experimental.pallas import tpu_sc as plsc

N = 10000
E = 320000
F_IN = 128
H = 128
C = 64

NPAD = 10240
NW = 32
CHW = 125
NCH = E // CHW
WCH = NCH // NW
PH0 = 40

_mesh = plsc.VectorSubcoreMesh(core_axis_name="c", subcore_axis_name="s")


@functools.partial(
    pl.kernel,
    mesh=_mesh,
    out_type=jax.ShapeDtypeStruct((2, 2, 1, NPAD), jnp.float32),
    scratch_types=[
        pltpu.VMEM((2, WCH, CHW), jnp.int32),
        pltpu.VMEM((CHW,), jnp.float32),
        pltpu.VMEM_SHARED((2 * NPAD,), jnp.float32),
        pltpu.SemaphoreType.DMA,
    ],
)
def _sc_degrees(idx_hbm, ones_hbm, zeros_hbm, out_hbm, idx_v, ones_v, acc_sh,
                sem):
    c = lax.axis_index("c")
    s = lax.axis_index("s")
    wid = s * 2 + c
    sl = (2 * NPAD) // 16
    pltpu.sync_copy(zeros_hbm.at[pl.ds(s * sl, sl)], acc_sh.at[pl.ds(s * sl, sl)])
    pltpu.sync_copy(ones_hbm, ones_v)
    pltpu.sync_copy(idx_hbm.at[0, pl.ds(wid * WCH, WCH)],
                    idx_v.at[0])
    pltpu.sync_copy(idx_hbm.at[1, pl.ds(wid * WCH, WCH)],
                    idx_v.at[1])
    plsc.subcore_barrier()

    for h in range(2):
        def body(j, carry, h=h):
            pltpu.async_copy(ones_v, acc_sh.at[idx_v.at[h, j]], sem, add=True)
            return carry

        lax.fori_loop(0, WCH, body, 0)

    def drain(j, carry):
        pltpu.make_async_copy(ones_v, acc_sh.at[idx_v.at[0, 0]], sem).wait()
        return carry

    lax.fori_loop(0, 2 * WCH, drain, 0)
    plsc.subcore_barrier()
    pltpu.sync_copy(
        acc_sh.at[pl.ds(s * sl, sl)],
        out_hbm.at[c, s // 8, 0, pl.ds((s % 8) * sl, sl)])


def _make_sc_agg(F, ph, chw=CHW, tc_tiling=True):
    kw = {}
    if not tc_tiling:
        kw["compiler_params"] = pltpu.CompilerParams(use_tc_tiling_on_sc=False)
    rows_shape = (chw, F)

    @functools.partial(
        pl.kernel,
        mesh=_mesh,
        out_type=jax.ShapeDtypeStruct((2, NPAD, F), jnp.float32),
        scratch_types=[
            pltpu.VMEM((ph, chw), jnp.int32),
            pltpu.VMEM((ph, chw), jnp.int32),
            pltpu.VMEM(rows_shape, jnp.float32),
            pltpu.VMEM(rows_shape, jnp.float32),
            pltpu.VMEM_SHARED((NPAD, F), jnp.float32),
            pltpu.SemaphoreType.DMA,
            pltpu.SemaphoreType.DMA,
        ],
        **kw,
    )
    def _sc_agg(table_hbm, sidx_hbm, didx_hbm, zeros_hbm, out_hbm,
                sidx_v, didx_v, rows0_v, rows1_v, acc_sh, g0, g1):
        c = lax.axis_index("c")
        s = lax.axis_index("s")
        wid = s * 2 + c
        rs = NPAD // 16
        pltpu.sync_copy(zeros_hbm.at[pl.ds(s * rs, rs)], acc_sh.at[pl.ds(s * rs, rs)])
        plsc.subcore_barrier()

        def sidx(j):
            return sidx_v.at[j]

        def didx(j):
            return didx_v.at[j]

        def ring(cpp):
            nd = cpp
            pltpu.async_copy(table_hbm.at[sidx(0)], rows0_v, g0)

            def body(j2, carry):
                j = j2 * 2
                pltpu.make_async_copy(table_hbm.at[sidx(j)], rows0_v,
                                      g0).wait()
                pltpu.async_copy(table_hbm.at[sidx(j + 1)], rows1_v, g1)
                pltpu.sync_copy(rows0_v, acc_sh.at[didx(j)], add=True)
                pltpu.make_async_copy(table_hbm.at[sidx(j + 1)], rows1_v,
                                      g1).wait()

                @pl.when(j2 < nd // 2 - 1)
                def _():
                    pltpu.async_copy(table_hbm.at[sidx(j + 2)], rows0_v, g0)

                pltpu.sync_copy(rows1_v, acc_sh.at[didx(j + 1)], add=True)
                return carry

            lax.fori_loop(0, nd // 2, body, 0)

        for p in range(WCH // ph):
            base = wid * WCH + p * ph
            pltpu.sync_copy(sidx_hbm.at[pl.ds(base, ph)], sidx_v)
            pltpu.sync_copy(didx_hbm.at[pl.ds(base, ph)], didx_v)
            ring(ph)

        plsc.subcore_barrier()
        pltpu.sync_copy(acc_sh.at[pl.ds(s * rs, rs)],
                        out_hbm.at[c, pl.ds(s * rs, rs)])

    return _sc_agg


_sc_agg_h = _make_sc_agg(H, PH0)
_sc_agg_c = _make_sc_agg(C, WCH, tc_tiling=False)


def _norm_col(deg_ref, which):
    d = deg_ref[0, which, 0, :] + deg_ref[1, which, 0, :]
    nrm = lax.rsqrt(jnp.maximum(d, 1.0))
    nb = jnp.broadcast_to(nrm.reshape(1, NPAD), (8, NPAD))
    return lax.transpose(nb, (1, 0))[0:N, 0:1]


def _tc_prep_body(x_ref, w_ref, deg_ref, o_ref):
    norm_src = _norm_col(deg_ref, 0)
    x = x_ref[...] * norm_src
    o_ref[...] = jnp.dot(x, w_ref[...], preferred_element_type=jnp.float32)


def _tc_mid_body(aggp_ref, deg_ref, b1_ref, w2_ref, o_ref):
    norm_src = _norm_col(deg_ref, 0)
    norm_dst = _norm_col(deg_ref, 1)
    agg = aggp_ref[0, 0:N, :] + aggp_ref[1, 0:N, :]
    h = jnp.maximum(agg * norm_dst + b1_ref[...], 0.0)
    o_ref[...] = jnp.dot(h * norm_src, w2_ref[...],
                         preferred_element_type=jnp.float32)


def _tc_fin_body(aggp_ref, deg_ref, b2_ref, o_ref):
    norm_dst = _norm_col(deg_ref, 1)
    agg = aggp_ref[0, 0:N, :] + aggp_ref[1, 0:N, :]
    o_ref[...] = agg * norm_dst + b2_ref[...]


def kernel(n_feats, edge_index, W1, b1, W2, b2):
    ei = edge_index.astype(jnp.int32)
    src_c = ei[0].reshape(NCH, CHW)
    dst_c = ei[1].reshape(NCH, CHW)
    deg_idx = (ei + jnp.array([[0], [NPAD]], jnp.int32)).reshape(2, NCH, CHW)

    ones_chunk = jnp.ones((CHW,), jnp.float32)
    zeros_deg = jnp.zeros((2 * NPAD,), jnp.float32)
    zeros_h = jnp.zeros((NPAD, H), jnp.float32)
    zeros_c = jnp.zeros((NPAD, C), jnp.float32)

    degp = _sc_degrees(deg_idx, ones_chunk, zeros_deg)

    xw = pl.pallas_call(
        _tc_prep_body,
        out_shape=jax.ShapeDtypeStruct((N, H), jnp.float32),
    )(n_feats, W1, degp)

    agg1p = _sc_agg_h(xw, src_c, dst_c, zeros_h)

    y = pl.pallas_call(
        _tc_mid_body,
        out_shape=jax.ShapeDtypeStruct((N, C), jnp.float32),
    )(agg1p, degp, b1.reshape(1, H), W2)

    agg2p = _sc_agg_c(y, src_c, dst_c, zeros_c)

    out = pl.pallas_call(
        _tc_fin_body,
        out_shape=jax.ShapeDtypeStruct((N, C), jnp.float32),
    )(agg2p, degp, b2.reshape(1, C))

    return out
